# baseline (speedup 1.0000x reference)
import functools

import jax
import jax.numpy as jnp
from jax import lax
from jax.experimental import pallas as pl
from jax.experimental.pallas import tpu as pltpu
from jax.experimental.pallas import tpu_sc as plsc

NUM_CORES = 2
NUM_SUBCORES = 16
NUM_WORKERS = NUM_CORES * NUM_SUBCORES
NBUF = 11


def _build(B, D, V):
    b_per_w = B // NUM_WORKERS

    mesh = plsc.VectorSubcoreMesh(
        core_axis_name="c",
        subcore_axis_name="s",
        num_cores=NUM_CORES,
        num_subcores=NUM_SUBCORES,
    )

    @functools.partial(
        pl.kernel,
        out_type=jax.ShapeDtypeStruct((B * D,), jnp.float32),
        mesh=mesh,
        scratch_types=[
            pltpu.VMEM_SHARED((NUM_SUBCORES, b_per_w), jnp.int32),
            pltpu.SMEM((b_per_w,), jnp.int32),
            pltpu.SMEM((b_per_w,), jnp.int32),
            pltpu.SMEM((b_per_w + 1,), jnp.int32),
            pltpu.VMEM((NBUF, 8, 8, 128), jnp.float32),
            pltpu.VMEM((b_per_w * D,), jnp.float32),
            pltpu.SemaphoreType.DMA,
            pltpu.SemaphoreType.DMA,
        ] + [pltpu.SemaphoreType.DMA] * NBUF,
        compiler_params=pltpu.CompilerParams(needs_layout_passes=False),
    )
    def embed(t3_hbm, slab_hbm, pos_hbm, out_hbm, stage_v, slab_s, pos_s,
              rstart_s, blocks_v, rows_v, lsem, wsem, *sems):
        wid = lax.axis_index("s") * NUM_CORES + lax.axis_index("c")
        base = wid * b_per_w
        sid = lax.axis_index("s")
        pltpu.async_copy(
            slab_hbm.at[pl.ds(base, b_per_w)], stage_v.at[sid], lsem).wait()
        pltpu.sync_copy(stage_v.at[sid], slab_s)
        pltpu.async_copy(
            pos_hbm.at[pl.ds(base, b_per_w)], stage_v.at[sid], lsem).wait()
        pltpu.sync_copy(stage_v.at[sid], pos_s)

        def scan(i, carry):
            nu, prev = carry
            t = lax.shift_right_logical(slab_s[i], 7)
            isnew = t != prev

            @pl.when(isnew)
            def _():
                rstart_s[nu] = i

            return (jnp.where(isnew, nu + 1, nu), t)

        nu, _ = lax.fori_loop(0, b_per_w, scan, (0, -1))
        rstart_s[nu] = b_per_w

        u = lax.iota(jnp.int32, 16)
        idx1 = lax.bitwise_and(u, 7)
        u8 = lax.shift_right_logical(u, 3)
        idx0s = [u8 + (2 * k) for k in range(4)]

        def fetch(j, slot):
            @pl.when(j < nu)
            def _():
                t = lax.shift_right_logical(slab_s[rstart_s[j]], 7)
                off = pl.multiple_of(lax.shift_left(t, 7), 128)
                pltpu.async_copy(
                    t3_hbm.at[:, :, pl.ds(off, 128)],
                    blocks_v.at[slot],
                    sems[slot],
                )

        def wait_slot(slot):
            pltpu.make_async_copy(
                t3_hbm.at[:, :, pl.ds(0, 128)],
                blocks_v.at[slot],
                sems[slot],
            ).wait()

        for b in range(NBUF):
            fetch(b, b)

        def body(g, _):
            for b in range(NBUF):
                j = g * NBUF + b

                @pl.when(j < nu)
                def _():
                    wait_slot(b)
                    blk = blocks_v.at[b]

                    def el(i, _c):
                        il = lax.bitwise_and(slab_s[i], 127)
                        idx2 = lax.broadcast(il, (16,))
                        for k in range(4):
                            vals = plsc.load_gather(
                                blk, [idx0s[k], idx1, idx2])
                            rows_v[pl.ds(i * D + k * 16, 16)] = vals
                        pltpu.async_copy(
                            rows_v.at[pl.ds(i * D, D)],
                            out_hbm.at[pl.ds(pos_s[i] * D, D)],
                            wsem,
                        )
                        return 0

                    lax.fori_loop(rstart_s[j], rstart_s[j + 1], el, 0)
                    fetch(j + NBUF, b)
            return 0

        ngroups = lax.div(nu + (NBUF - 1), NBUF)
        lax.fori_loop(0, ngroups, body, 0)
        pltpu.make_async_copy(
            out_hbm.at[pl.ds(base * D, b_per_w * D)], rows_v, wsem).wait()

    return embed


def kernel(labels, embedding_table):
    B = labels.shape[0]
    V, D = embedding_table.shape
    lab = labels.astype(jnp.int32)
    pos = jnp.argsort(lab).astype(jnp.int32)
    slab = jnp.take(lab, pos)
    t3 = jnp.swapaxes(embedding_table, 0, 1).reshape(D // 8, 8, V)
    out_flat = _build(B, D, V)(t3, slab, pos)
    return out_flat.reshape(B, D)


# speedup vs baseline: 1.0133x; 1.0133x over previous
import functools

import jax
import jax.numpy as jnp
from jax import lax
from jax.experimental import pallas as pl
from jax.experimental.pallas import tpu as pltpu
from jax.experimental.pallas import tpu_sc as plsc

NUM_CORES = 2
NUM_SUBCORES = 16
NUM_WORKERS = NUM_CORES * NUM_SUBCORES
NBUF = 8


def _build(B, D, V):
    b_per_w = B // NUM_WORKERS

    mesh = plsc.VectorSubcoreMesh(
        core_axis_name="c",
        subcore_axis_name="s",
        num_cores=NUM_CORES,
        num_subcores=NUM_SUBCORES,
    )

    @functools.partial(
        pl.kernel,
        out_type=jax.ShapeDtypeStruct((B * D,), jnp.float32),
        mesh=mesh,
        scratch_types=[
            pltpu.VMEM_SHARED((NUM_SUBCORES, b_per_w), jnp.int32),
            pltpu.SMEM((b_per_w,), jnp.int32),
            pltpu.SMEM((b_per_w,), jnp.int32),
            pltpu.SMEM((b_per_w + 1,), jnp.int32),
            pltpu.VMEM((NBUF, 8, 8, 128), jnp.float32),
            pltpu.VMEM((b_per_w * D,), jnp.float32),
            pltpu.SemaphoreType.DMA,
            pltpu.SemaphoreType.DMA,
        ] + [pltpu.SemaphoreType.DMA] * NBUF,
        compiler_params=pltpu.CompilerParams(needs_layout_passes=False),
    )
    def embed(t3_hbm, slab_hbm, pos_hbm, out_hbm, stage_v, slab_s, pos_s,
              rstart_s, blocks_v, rows_v, lsem, wsem, *sems):
        wid = lax.axis_index("s") * NUM_CORES + lax.axis_index("c")
        base = wid * b_per_w
        sid = lax.axis_index("s")
        pltpu.async_copy(
            slab_hbm.at[pl.ds(base, b_per_w)], stage_v.at[sid], lsem).wait()
        pltpu.sync_copy(stage_v.at[sid], slab_s)
        pltpu.async_copy(
            pos_hbm.at[pl.ds(base, b_per_w)], stage_v.at[sid], lsem).wait()
        pltpu.sync_copy(stage_v.at[sid], pos_s)

        def scan(i, carry):
            nu, prev = carry
            t = lax.shift_right_logical(slab_s[i], 7)
            isnew = t != prev

            @pl.when(isnew)
            def _():
                rstart_s[nu] = i

            return (jnp.where(isnew, nu + 1, nu), t)

        nu, _ = lax.fori_loop(0, b_per_w, scan, (0, -1))
        rstart_s[nu] = b_per_w

        u = lax.iota(jnp.int32, 16)
        idx1 = lax.bitwise_and(u, 7)
        u8 = lax.shift_right_logical(u, 3)
        idx0s = [u8 + (2 * k) for k in range(4)]

        def fetch(j, slot):
            @pl.when(j < nu)
            def _():
                t = lax.shift_right_logical(slab_s[rstart_s[j]], 7)
                off = pl.multiple_of(lax.shift_left(t, 7), 128)
                pltpu.async_copy(
                    t3_hbm.at[:, :, pl.ds(off, 128)],
                    blocks_v.at[slot],
                    sems[slot],
                )

        def wait_slot(slot):
            pltpu.make_async_copy(
                t3_hbm.at[:, :, pl.ds(0, 128)],
                blocks_v.at[slot],
                sems[slot],
            ).wait()

        for b in range(NBUF):
            fetch(b, b)

        def body(g, _):
            for b in range(NBUF):
                j = g * NBUF + b

                @pl.when(j < nu)
                def _():
                    wait_slot(b)
                    blk = blocks_v.at[b]

                    def el(i, _c):
                        il = lax.bitwise_and(slab_s[i], 127)
                        idx2 = lax.broadcast(il, (16,))
                        for k in range(4):
                            vals = plsc.load_gather(
                                blk, [idx0s[k], idx1, idx2])
                            rows_v[pl.ds(i * D + k * 16, 16)] = vals
                        pltpu.async_copy(
                            rows_v.at[pl.ds(i * D, D)],
                            out_hbm.at[pl.ds(pos_s[i] * D, D)],
                            wsem,
                        )
                        return 0

                    lax.fori_loop(rstart_s[j], rstart_s[j + 1], el, 0)
                    fetch(j + NBUF, b)
            return 0

        ngroups = lax.div(nu + (NBUF - 1), NBUF)
        lax.fori_loop(0, ngroups, body, 0)
        pltpu.make_async_copy(
            out_hbm.at[pl.ds(base * D, b_per_w * D)], rows_v, wsem).wait()

    return embed


def kernel(labels, embedding_table):
    B = labels.shape[0]
    V, D = embedding_table.shape
    lab = labels.astype(jnp.int32)
    pos = jnp.argsort(lab).astype(jnp.int32)
    slab = jnp.take(lab, pos)
    t3 = jnp.swapaxes(embedding_table, 0, 1).reshape(D // 8, 8, V)
    out_flat = _build(B, D, V)(t3, slab, pos)
    return out_flat.reshape(B, D)
